# Initial kernel scaffold; baseline (speedup 1.0000x reference)
import jax
import jax.numpy as jnp
from jax.experimental import pallas as pl


def kernel(X_user_avg_log, X_hist_liked, X_hist_disliked, X_hist_full, X_hist_playtime_weights, target_year_idx, target_game_idx, target_dev_idx, target_price, item_table, dev_table, year_table, price_table, W_item_t, b_item_t, W_dev_t, b_dev_t, W_tag1, b_tag1, W_tag2, b_tag2, W_ig, b_ig, W_yr, b_yr, W_pr, b_pr, W_ug1, b_ug1, W_ug2, b_ug2, W_ut1, b_ut1, W_ut2, b_ut2, W_up1, b_up1, W_up2, b_up2, W_ip1, b_ip1, W_ip2, b_ip2, game_tag_matrix, game_genre_matrix):
    raise NotImplementedError("write your pallas kernel here")


# speedup vs baseline: 5.2916x; 5.2916x over previous
import jax
import jax.numpy as jnp
from jax import lax
from jax.experimental import pallas as pl
from jax.experimental.pallas import tpu as pltpu
from jax.experimental.pallas import tpu_sc as plsc

_B = 4096
_NGAMES = 100000
_NDEVS = 50000
_NC, _NS, _L = 2, 16, 16
_NW = _NC * _NS
_BPW = _B // _NW

_LF = 200
_LFA, _LFB = 128, 80
_LFP = _LFA + _LFB
_LL, _LLP = 50, 64
_LD, _LDP = 20, 32

_PW = 320


_DNUMS = lax.GatherDimensionNumbers(offset_dims=(), collapsed_slice_dims=(0,),
                                    start_index_map=(0,))


def _splat(v, i):
    idx = jnp.full((_L, 1), i, jnp.int32)
    return lax.gather(v, idx, _DNUMS, (1,),
                      mode=lax.GatherScatterMode.PROMISE_IN_BOUNDS)


def _sc_body(item_tbl, genre_mat, tag_mat, dev_tbl,
             idx_full, idx_liked, idx_dis, w_full, tgt_game, tgt_dev,
             pooled_out, titem_out, tdev_out, tgenre_out, ttag_out,
             idxfa_v, idxfb_v, idxl_v, idxd_v, w_v,
             item_rows, genre_rows, tag_rows, liked_rows, dis_rows,
             tgi_v, tdi_v, trows_item, trows_dev, trows_genre, trows_tag,
             pooled_buf, sem):
    wid = lax.axis_index("s") * _NC + lax.axis_index("c")
    base = wid * _BPW
    pad16 = jnp.full((_L,), _NGAMES, jnp.int32)
    zero = jnp.zeros((_L,), jnp.float32)
    one = jnp.ones((_L,), jnp.float32)

    pltpu.sync_copy(tgt_game.at[pl.ds(base, _BPW)], tgi_v)
    pltpu.sync_copy(tgt_dev.at[pl.ds(base, _BPW)], tdi_v)
    c1 = pltpu.async_copy(item_tbl.at[tgi_v], trows_item, sem)
    c2 = pltpu.async_copy(dev_tbl.at[tdi_v], trows_dev, sem)
    c3 = pltpu.async_copy(genre_mat.at[tgi_v], trows_genre, sem)
    c4 = pltpu.async_copy(tag_mat.at[tgi_v], trows_tag, sem)
    c1.wait(); c2.wait(); c3.wait(); c4.wait()
    pltpu.sync_copy(trows_item, titem_out.at[pl.ds(base, _BPW)])
    pltpu.sync_copy(trows_dev, tdev_out.at[pl.ds(base, _BPW)])
    pltpu.sync_copy(trows_genre, tgenre_out.at[pl.ds(base, _BPW)])
    pltpu.sync_copy(trows_tag, ttag_out.at[pl.ds(base, _BPW)])

    idxfb_v[pl.ds(_LFB - _L, _L)] = pad16
    idxl_v[pl.ds(_LLP - _L, _L)] = pad16
    idxd_v[pl.ds(_LDP - _L, _L)] = pad16

    def row_body(r, _):
        b = base + r
        pltpu.sync_copy(idx_full.at[b, pl.ds(0, _LFA)], idxfa_v)
        pltpu.sync_copy(idx_full.at[b, pl.ds(_LFA, _LF - _LFA)],
                        idxfb_v.at[pl.ds(0, _LF - _LFA)])
        pltpu.sync_copy(idx_liked.at[b], idxl_v.at[pl.ds(0, _LL)])
        pltpu.sync_copy(idx_dis.at[b], idxd_v.at[pl.ds(0, _LD)])
        pltpu.sync_copy(w_full.at[b], w_v.at[pl.ds(0, _LF)])

        cps = [
            pltpu.async_copy(item_tbl.at[idxfa_v],
                             item_rows.at[pl.ds(0, _LFA)], sem),
            pltpu.async_copy(item_tbl.at[idxfb_v],
                             item_rows.at[pl.ds(_LFA, _LFB)], sem),
            pltpu.async_copy(genre_mat.at[idxfa_v],
                             genre_rows.at[pl.ds(0, _LFA)], sem),
            pltpu.async_copy(genre_mat.at[idxfb_v],
                             genre_rows.at[pl.ds(_LFA, _LFB)], sem),
            pltpu.async_copy(tag_mat.at[idxfa_v],
                             tag_rows.at[pl.ds(0, _LFA)], sem),
            pltpu.async_copy(tag_mat.at[idxfb_v],
                             tag_rows.at[pl.ds(_LFA, _LFB)], sem),
            pltpu.async_copy(item_tbl.at[idxl_v], liked_rows, sem),
            pltpu.async_copy(item_tbl.at[idxd_v], dis_rows, sem),
        ]

        for cp in cps:
            cp.wait()

        def full_chunk(o, jbase, iv, wv, acc):
            (f0, f1, p0, p1, c0, c1_, g0, g1,
             t0, t1, t2, t3, t4, t5, t6, t7) = acc
            m = iv != _NGAMES
            m1c = jnp.where(m, one, zero)
            wmc = jnp.where(m, wv, zero)
            for i in range(_L):
                j = jbase + i
                m1s = _splat(m1c, i)
                wms = _splat(wmc, i)
                i0 = item_rows[j, pl.ds(0, _L)]
                i1 = item_rows[j, pl.ds(_L, _L)]
                q0 = genre_rows[j, pl.ds(0, _L)]
                q1 = genre_rows[j, pl.ds(_L, _L)]
                f0 = f0 + m1s * i0
                f1 = f1 + m1s * i1
                p0 = p0 + wms * i0
                p1 = p1 + wms * i1
                c0 = c0 + jnp.where(q0 > 0, one, zero)
                c1_ = c1_ + jnp.where(q1 > 0, one, zero)
                g0 = g0 + wms * q0
                g1 = g1 + wms * q1
                t0 = t0 + tag_rows[j, pl.ds(0, _L)]
                t1 = t1 + tag_rows[j, pl.ds(16, _L)]
                t2 = t2 + tag_rows[j, pl.ds(32, _L)]
                t3 = t3 + tag_rows[j, pl.ds(48, _L)]
                t4 = t4 + tag_rows[j, pl.ds(64, _L)]
                t5 = t5 + tag_rows[j, pl.ds(80, _L)]
                t6 = t6 + tag_rows[j, pl.ds(96, _L)]
                t7 = t7 + tag_rows[j, pl.ds(112, _L)]
            return (f0, f1, p0, p1, c0, c1_, g0, g1,
                    t0, t1, t2, t3, t4, t5, t6, t7)

        def chunk_a(c, acc):
            o = pl.multiple_of(c * _L, _L)
            return full_chunk(o, o, idxfa_v[pl.ds(o, _L)],
                              w_v[pl.ds(o, _L)], acc)

        def chunk_b(c, acc):
            o = pl.multiple_of(c * _L, _L)
            return full_chunk(o, _LFA + o, idxfb_v[pl.ds(o, _L)],
                              w_v[pl.ds(_LFA + o, _L)], acc)

        acc = lax.fori_loop(0, _LFA // _L, chunk_a, (zero,) * 16)
        acc = lax.fori_loop(0, _LFB // _L, chunk_b, acc)
        (f0, f1, p0, p1, c0, c1_, g0, g1,
         t0, t1, t2, t3, t4, t5, t6, t7) = acc

        def chunk_l(c, a):
            l0, l1 = a
            o = pl.multiple_of(c * _L, _L)
            m1c = jnp.where(idxl_v[pl.ds(o, _L)] != _NGAMES, one, zero)
            for i in range(_L):
                m1s = _splat(m1c, i)
                l0 = l0 + m1s * liked_rows[o + i, pl.ds(0, _L)]
                l1 = l1 + m1s * liked_rows[o + i, pl.ds(_L, _L)]
            return (l0, l1)
        l0, l1 = lax.fori_loop(0, _LLP // _L, chunk_l, (zero, zero))

        def chunk_d(c, a):
            d0, d1 = a
            o = pl.multiple_of(c * _L, _L)
            m1c = jnp.where(idxd_v[pl.ds(o, _L)] != _NGAMES, one, zero)
            for i in range(_L):
                m1s = _splat(m1c, i)
                d0 = d0 + m1s * dis_rows[o + i, pl.ds(0, _L)]
                d1 = d1 + m1s * dis_rows[o + i, pl.ds(_L, _L)]
            return (d0, d1)
        d0, d1 = lax.fori_loop(0, _LDP // _L, chunk_d, (zero, zero))

        pooled_buf[pl.ds(0, _L)] = f0
        pooled_buf[pl.ds(16, _L)] = f1
        pooled_buf[pl.ds(32, _L)] = p0
        pooled_buf[pl.ds(48, _L)] = p1
        pooled_buf[pl.ds(64, _L)] = c0
        pooled_buf[pl.ds(80, _L)] = c1_
        pooled_buf[pl.ds(96, _L)] = g0
        pooled_buf[pl.ds(112, _L)] = g1
        pooled_buf[pl.ds(128, _L)] = l0
        pooled_buf[pl.ds(144, _L)] = l1
        pooled_buf[pl.ds(160, _L)] = d0
        pooled_buf[pl.ds(176, _L)] = d1
        pooled_buf[pl.ds(192, _L)] = t0
        pooled_buf[pl.ds(208, _L)] = t1
        pooled_buf[pl.ds(224, _L)] = t2
        pooled_buf[pl.ds(240, _L)] = t3
        pooled_buf[pl.ds(256, _L)] = t4
        pooled_buf[pl.ds(272, _L)] = t5
        pooled_buf[pl.ds(288, _L)] = t6
        pooled_buf[pl.ds(304, _L)] = t7
        pltpu.sync_copy(pooled_buf, pooled_out.at[b])
        return 0

    lax.fori_loop(0, _BPW, row_body, 0)


def _make_sc():
    mesh = plsc.VectorSubcoreMesh(core_axis_name="c", subcore_axis_name="s")
    f32, i32 = jnp.float32, jnp.int32
    return pl.kernel(
        _sc_body,
        out_type=(
            jax.ShapeDtypeStruct((_B, _PW), f32),
            jax.ShapeDtypeStruct((_B, 32), f32),
            jax.ShapeDtypeStruct((_B, 16), f32),
            jax.ShapeDtypeStruct((_B, 32), f32),
            jax.ShapeDtypeStruct((_B, 128), f32),
        ),
        mesh=mesh,
        compiler_params=pltpu.CompilerParams(use_tc_tiling_on_sc=False),
        scratch_types=[
            pltpu.VMEM((_LFA,), i32),
            pltpu.VMEM((_LFB,), i32),
            pltpu.VMEM((_LLP,), i32),
            pltpu.VMEM((_LDP,), i32),
            pltpu.VMEM((_LFP,), f32),
            pltpu.VMEM((_LFP, 32), f32),
            pltpu.VMEM((_LFP, 32), f32),
            pltpu.VMEM((_LFP, 128), f32),
            pltpu.VMEM((_LLP, 32), f32),
            pltpu.VMEM((_LDP, 32), f32),
            pltpu.VMEM((_BPW,), i32),
            pltpu.VMEM((_BPW,), i32),
            pltpu.VMEM((_BPW, 32), f32),
            pltpu.VMEM((_BPW, 16), f32),
            pltpu.VMEM((_BPW, 32), f32),
            pltpu.VMEM((_BPW, 128), f32),
            pltpu.VMEM((_PW,), f32),
            pltpu.SemaphoreType.DMA,
        ],
    )


def _mm(a, b):
    return lax.dot_general(a, b, (((1,), (0,)), ((), ())),
                           precision=lax.Precision.HIGHEST,
                           preferred_element_type=jnp.float32)


def _relu(x):
    return jnp.maximum(x, 0.0)


_TCB = 512


def _tc_body(pooled, hist_full, ualog, titem, tdev, tgenre, ttag,
             tg_idx, td_idx, ty_idx, tp_idx, year_tbl, price_tbl,
             wug1, bug1, wug2, bug2, wut1, but1, wut2, but2,
             wup1, bup1, wup2, bup2,
             wig, big, wtag1, btag1, wtag2, btag2,
             witem, bitem, wdev, bdev, wyr, byr, wpr, bpr,
             wip1, bip1, wip2, bip2, out_ref):
    x = pooled[...]
    full = x[:, 0:32]
    play = x[:, 32:64]
    cnt = x[:, 64:96]
    gw = x[:, 96:128]
    liked = x[:, 128:160]
    dis = x[:, 160:192]
    xtag = x[:, 192:320]

    hf = hist_full[...]
    nv = jnp.sum(jnp.where(hf != _NGAMES, 1.0, 0.0).astype(jnp.float32),
                 axis=1, keepdims=True)
    ual = ualog[...]
    safe_cnt = jnp.where(cnt > 0, cnt, 1.0)
    aff = jnp.where(cnt > 0, ual * (nv * gw / safe_cnt - 1.0), 0.0)
    tot = jnp.sum(cnt, axis=1, keepdims=True)
    frac = cnt / jnp.where(tot > 0, tot, 1.0)

    w1 = wug1[...]
    h = _relu(_mm(aff, w1[0:32, :]) + _mm(frac, w1[32:64, :]) + bug1[...])
    genre_emb = _relu(_mm(h, wug2[...]) + bug2[...])

    h = _relu(_mm(xtag, wut1[...]) + but1[...])
    tag_emb = _relu(_mm(h, wut2[...]) + but2[...])

    wu = wup1[...]
    h = _relu(_mm(liked, wu[0:32, :]) + _mm(dis, wu[32:64, :]) +
              _mm(full, wu[64:96, :]) + _mm(play, wu[96:128, :]) +
              _mm(genre_emb, wu[128:160, :]) + _mm(tag_emb, wu[160:192, :]) +
              bup1[...])
    yu = _mm(h, wup2[...]) + bup2[...]

    ig_emb = _relu(_mm(tgenre[...], wig[...]) + big[...])
    h = _relu(_mm(ttag[...], wtag1[...]) + btag1[...])
    itag_emb = _relu(_mm(h, wtag2[...]) + btag2[...])
    item_row = jnp.where(tg_idx[...] == _NGAMES, 0.0, titem[...])
    iid_emb = _relu(_mm(item_row, witem[...]) + bitem[...])
    dev_row = jnp.where(td_idx[...] == _NDEVS, 0.0, tdev[...][:, 0:12])
    dev_emb = _relu(_mm(dev_row, wdev[...]) + bdev[...])
    yoh = (ty_idx[...] == lax.broadcasted_iota(jnp.int32, (1, 50), 1)
           ).astype(jnp.float32)
    yemb = _relu(_mm(_mm(yoh, year_tbl[...]), wyr[...]) + byr[...])
    poh = (tp_idx[...] == lax.broadcasted_iota(jnp.int32, (1, 20), 1)
           ).astype(jnp.float32)
    pemb = _relu(_mm(_mm(poh, price_tbl[...]), wpr[...]) + bpr[...])

    wi = wip1[...]
    h = _relu(_mm(ig_emb, wi[0:8, :]) + _mm(itag_emb, wi[8:24, :]) +
              _mm(iid_emb, wi[24:56, :]) + _mm(dev_emb, wi[56:68, :]) +
              _mm(yemb, wi[68:76, :]) + _mm(pemb, wi[76:80, :]) + bip1[...])
    yi = _mm(h, wip2[...]) + bip2[...]

    nu = jnp.maximum(jnp.sqrt(jnp.sum(yu * yu, axis=1, keepdims=True)), 1e-12)
    ni = jnp.maximum(jnp.sqrt(jnp.sum(yi * yi, axis=1, keepdims=True)), 1e-12)
    s = jnp.sum(yu * yi, axis=1, keepdims=True)
    out_ref[...] = s / (nu * ni)


def _make_tc():
    f32 = jnp.float32
    row = lambda i: (i, 0)
    rep = lambda i: (0, 0)

    def bs(shape, m):
        return pl.BlockSpec(shape, m)

    in_specs = [
        bs((_TCB, _PW), row),
        bs((_TCB, _LF), row),
        bs((_TCB, 1), row),
        bs((_TCB, 32), row),
        bs((_TCB, 16), row),
        bs((_TCB, 32), row),
        bs((_TCB, 128), row),
        bs((_TCB, 1), row),
        bs((_TCB, 1), row),
        bs((_TCB, 1), row),
        bs((_TCB, 1), row),
        bs((50, 8), rep),
        bs((20, 4), rep),
        bs((64, 128), rep), bs((1, 128), rep),
        bs((128, 32), rep), bs((1, 32), rep),
        bs((128, 256), rep), bs((1, 256), rep),
        bs((256, 32), rep), bs((1, 32), rep),
        bs((192, 256), rep), bs((1, 256), rep),
        bs((256, 128), rep), bs((1, 128), rep),
        bs((32, 8), rep), bs((1, 8), rep),
        bs((128, 128), rep), bs((1, 128), rep),
        bs((128, 16), rep), bs((1, 16), rep),
        bs((32, 32), rep), bs((1, 32), rep),
        bs((12, 12), rep), bs((1, 12), rep),
        bs((8, 8), rep), bs((1, 8), rep),
        bs((4, 4), rep), bs((1, 4), rep),
        bs((80, 256), rep), bs((1, 256), rep),
        bs((256, 128), rep), bs((1, 128), rep),
    ]
    return pl.pallas_call(
        _tc_body,
        grid=(_B // _TCB,),
        in_specs=in_specs,
        out_specs=pl.BlockSpec((_TCB, 1), row),
        out_shape=jax.ShapeDtypeStruct((_B, 1), f32),
    )


def kernel(X_user_avg_log, X_hist_liked, X_hist_disliked, X_hist_full,
           X_hist_playtime_weights, target_year_idx, target_game_idx,
           target_dev_idx, target_price, item_table, dev_table, year_table,
           price_table, W_item_t, b_item_t, W_dev_t, b_dev_t, W_tag1, b_tag1,
           W_tag2, b_tag2, W_ig, b_ig, W_yr, b_yr, W_pr, b_pr, W_ug1, b_ug1,
           W_ug2, b_ug2, W_ut1, b_ut1, W_ut2, b_ut2, W_up1, b_up1, W_up2,
           b_up2, W_ip1, b_ip1, W_ip2, b_ip2, game_tag_matrix,
           game_genre_matrix):
    i32 = jnp.int32
    idx_full = X_hist_full.astype(i32)
    idx_liked = X_hist_liked.astype(i32)
    idx_dis = X_hist_disliked.astype(i32)
    tg = target_game_idx.astype(i32)
    td = target_dev_idx.astype(i32)
    ty = target_year_idx.astype(i32)
    tp = target_price.astype(i32)
    dev_pad = jnp.concatenate(
        [dev_table, jnp.zeros((dev_table.shape[0], 4), jnp.float32)], axis=1)

    sc = _make_sc()
    pooled, titem, tdev, tgenre, ttag = sc(
        item_table, game_genre_matrix, game_tag_matrix, dev_pad,
        idx_full, idx_liked, idx_dis, X_hist_playtime_weights, tg, td)

    tc = _make_tc()
    out = tc(pooled, idx_full, X_user_avg_log, titem, tdev, tgenre, ttag,
             tg[:, None], td[:, None], ty[:, None], tp[:, None],
             year_table, price_table,
             W_ug1, b_ug1[None, :], W_ug2, b_ug2[None, :],
             W_ut1, b_ut1[None, :], W_ut2, b_ut2[None, :],
             W_up1, b_up1[None, :], W_up2, b_up2[None, :],
             W_ig, b_ig[None, :], W_tag1, b_tag1[None, :],
             W_tag2, b_tag2[None, :], W_item_t, b_item_t[None, :],
             W_dev_t, b_dev_t[None, :], W_yr, b_yr[None, :],
             W_pr, b_pr[None, :], W_ip1, b_ip1[None, :],
             W_ip2, b_ip2[None, :])
    return out[:, 0]
